# baseline (speedup 1.0000x reference)
import functools

import jax
import jax.numpy as jnp
from jax import lax
from jax.experimental import pallas as pl
from jax.experimental.pallas import tpu as pltpu
from jax.experimental.pallas import tpu_sc as plsc

N = 10000
D = 256
E = 160000
NT = 32
WIDTH = 320
NPAD = NT * WIDTH
CAP = 6144
NHIST = 10240
ECH = 2000
EPT = E // NT
EP = 163840
EPS = EP // NT
G = 32
GS = 64

_SC_PARAMS = pltpu.CompilerParams(needs_layout_passes=False)


def _mesh():
    return plsc.VectorSubcoreMesh(core_axis_name="c", subcore_axis_name="s")


def _wid():
    return lax.axis_index("s") * 2 + lax.axis_index("c")


_LANES = lambda: lax.iota(jnp.int32, 16)


def _mm3_body(a_ref, w0, w1, w2, b, o0, o1, o2, *, relu):
    t = a_ref[...] + b[...]
    if relu:
        t = jnp.maximum(t, 0.0)
    o0[...] = jnp.dot(t, w0[...], preferred_element_type=jnp.float32)
    o1[...] = jnp.dot(t, w1[...], preferred_element_type=jnp.float32)
    o2[...] = jnp.dot(t, w2[...], preferred_element_type=jnp.float32)


def _mm3(a, w0, w1, w2, b, relu):
    rows = NPAD // 8
    blk = pl.BlockSpec((rows, D), lambda i: (i, 0))
    wspec = pl.BlockSpec((D, D), lambda i: (0, 0))
    bspec = pl.BlockSpec((1, D), lambda i: (0, 0))
    sds = jax.ShapeDtypeStruct((NPAD, D), jnp.float32)
    return pl.pallas_call(
        functools.partial(_mm3_body, relu=relu),
        grid=(8,),
        in_specs=[blk, wspec, wspec, wspec, bspec],
        out_specs=(blk, blk, blk),
        out_shape=(sds, sds, sds),
    )(a, w0, w1, w2, b)


def _norm_body(dpart_ref, o_ref):
    deg = jnp.sum(dpart_ref[0], axis=0)
    nrm = jnp.where(deg > 0, lax.rsqrt(deg), 0.0)
    o_ref[...] = jnp.broadcast_to(nrm[None, :], o_ref.shape)


def _norms(deg_part):
    return pl.pallas_call(
        _norm_body,
        grid=(6,),
        in_specs=[pl.BlockSpec((1, NT, NHIST), lambda i: (i, 0, 0))],
        out_specs=pl.BlockSpec((8, NHIST), lambda i: (i, 0)),
        out_shape=jax.ShapeDtypeStruct((48, NHIST), jnp.float32),
    )(deg_part)


def _t2_body(h_ref, b_ref, o_ref):
    b = b_ref[...]
    t2 = jnp.sum(h_ref[...] * b, axis=1) + 0.5 * jnp.sum(b * b)
    o_ref[...] = jnp.broadcast_to(t2[None, :], o_ref.shape)


def _t2(h2_pre, b2sum):
    return pl.pallas_call(
        _t2_body,
        out_shape=jax.ShapeDtypeStruct((8, NPAD), jnp.float32),
    )(h2_pre, b2sum)


def _sc_degree(s0, d0, s1, d1, s2, d2):
    out_type = jax.ShapeDtypeStruct((6 * NT * NHIST,), jnp.float32)

    @functools.partial(
        pl.kernel, mesh=_mesh(), out_type=out_type,
        compiler_params=_SC_PARAMS,
        scratch_types=[
            pltpu.VMEM((NHIST,), jnp.float32),
            pltpu.VMEM((5008,), jnp.int32),
        ],
    )
    def k(e0, e1, e2, e3, e4, e5, o, hist_v, idx_v):
        wid = _wid()
        lanes = _LANES()
        ones = jnp.ones((16,), jnp.float32)
        z = jnp.zeros((16,), jnp.float32)
        for ai, eref in enumerate((e0, e1, e2, e3, e4, e5)):
            def zbody(i, _):
                hist_v[pl.ds(i * 16, 16)] = z
                return 0
            lax.fori_loop(0, NHIST // 16, zbody, 0, unroll=8)
            pltpu.sync_copy(eref.at[pl.ds(wid * EPT, EPT)], idx_v.at[pl.ds(0, EPT)])

            def hbody(i, _):
                v = idx_v[pl.ds(i * 16, 16)]
                plsc.addupdate_scatter(hist_v, [v], ones)
                return 0
            lax.fori_loop(0, EPT // 16, hbody, 0, unroll=8)
            vt = idx_v[pl.ds((EPT // 16) * 16, 16)]
            plsc.addupdate_scatter(hist_v, [vt], ones, mask=lanes < (EPT % 16))
            pltpu.sync_copy(hist_v, o.at[pl.ds((ai * NT + wid) * NHIST, NHIST)])

    return k(s0, d0, s1, d1, s2, d2)


def _sc_bin(s0, d0, s1, d1, s2, d2, norms):
    out_type = (
        jax.ShapeDtypeStruct((3 * NT * CAP,), jnp.int32),
        jax.ShapeDtypeStruct((3 * NT * CAP,), jnp.int32),
        jax.ShapeDtypeStruct((3 * NT * CAP,), jnp.float32),
        jax.ShapeDtypeStruct((3 * NT * 16,), jnp.int32),
    )

    @functools.partial(
        pl.kernel, mesh=_mesh(), out_type=out_type,
        compiler_params=_SC_PARAMS,
        scratch_types=[
            pltpu.VMEM((NHIST,), jnp.float32),
            pltpu.VMEM((NHIST,), jnp.float32),
            pltpu.VMEM((ECH,), jnp.int32),
            pltpu.VMEM((ECH,), jnp.int32),
            pltpu.VMEM((CAP + 16,), jnp.int32),
            pltpu.VMEM((CAP + 16,), jnp.int32),
            pltpu.VMEM((CAP + 16,), jnp.float32),
            pltpu.VMEM((16,), jnp.int32),
        ],
    )
    def k(es0, ed0, es1, ed1, es2, ed2, nrm,
          bsrc, bdl, bw, bcnt,
          ns_v, nd_v, sc_v, dc_v, os_v, od_v, ow_v, cnt_v):
        wid = _wid()
        zi = jnp.zeros((16,), jnp.int32)
        zf = jnp.zeros((16,), jnp.float32)
        srcs = (es0, es1, es2)
        dsts = (ed0, ed1, ed2)
        for r in range(3):
            pltpu.sync_copy(nrm.at[pl.ds((2 * r) * 8 * NHIST, NHIST)], ns_v)
            pltpu.sync_copy(nrm.at[pl.ds((2 * r + 1) * 8 * NHIST, NHIST)], nd_v)

            def zbody(i, _):
                os_v[pl.ds(i * 16, 16)] = zi
                od_v[pl.ds(i * 16, 16)] = zi
                ow_v[pl.ds(i * 16, 16)] = zf
                return 0
            lax.fori_loop(0, (CAP + 16) // 16, zbody, 0, unroll=8)

            def chunk(kk, cnt):
                pltpu.sync_copy(srcs[r].at[pl.ds(kk * ECH, ECH)], sc_v)
                pltpu.sync_copy(dsts[r].at[pl.ds(kk * ECH, ECH)], dc_v)

                def vbody(i, cnt):
                    d_v = dc_v[pl.ds(i * 16, 16)]
                    s_v = sc_v[pl.ds(i * 16, 16)]
                    q = (d_v * 3277) >> 20
                    q = jnp.where(q * WIDTH > d_v, q - 1, q)
                    m = (q == wid) & (cnt < CAP - 32)
                    dl = d_v - wid * WIDTH
                    w = plsc.load_gather(ns_v, [s_v]) * plsc.load_gather(nd_v, [d_v])
                    plsc.store_compressed(os_v.at[pl.ds(cnt, 16)], s_v, mask=m)
                    plsc.store_compressed(od_v.at[pl.ds(cnt, 16)], dl, mask=m)
                    plsc.store_compressed(ow_v.at[pl.ds(cnt, 16)], w, mask=m)
                    return cnt + plsc.all_reduce_population_count(m)[0]
                return lax.fori_loop(0, ECH // 16, vbody, cnt)

            cnt = lax.fori_loop(0, E // ECH, chunk, jnp.int32(0))
            os_v[pl.ds(cnt, 16)] = zi
            od_v[pl.ds(cnt, 16)] = zi
            ow_v[pl.ds(cnt, 16)] = zf
            pltpu.sync_copy(os_v.at[pl.ds(0, CAP)],
                            bsrc.at[pl.ds((r * NT + wid) * CAP, CAP)])
            pltpu.sync_copy(od_v.at[pl.ds(0, CAP)],
                            bdl.at[pl.ds((r * NT + wid) * CAP, CAP)])
            pltpu.sync_copy(ow_v.at[pl.ds(0, CAP)],
                            bw.at[pl.ds((r * NT + wid) * CAP, CAP)])
            cnt_v[...] = jnp.broadcast_to(cnt, (16,))
            pltpu.sync_copy(cnt_v, bcnt.at[pl.ds((r * NT + wid) * 16, 16)])

    return k(s0, d0, s1, d1, s2, d2, norms)


def _sc_spmm(y0, y1, y2, bsrc, bdl, bw, bcnt):
    out_type = jax.ShapeDtypeStruct((NPAD, D), jnp.float32)

    @functools.partial(
        pl.kernel, mesh=_mesh(), out_type=out_type,
        compiler_params=_SC_PARAMS,
        scratch_types=[
            pltpu.VMEM((WIDTH, D), jnp.float32),
            pltpu.VMEM((3 * G, D), jnp.float32),
            pltpu.VMEM((CAP,), jnp.int32),
            pltpu.VMEM((CAP,), jnp.int32),
            pltpu.VMEM((CAP,), jnp.float32),
            pltpu.VMEM((16,), jnp.int32),
            pltpu.SemaphoreType.DMA,
        ],
    )
    def k(ya, yb, yc, bs, bd, bww, bc, o,
          acc_v, rows_v, s_v, d_v, w_v, cnt_v, sem_g):
        wid = _wid()
        z = jnp.zeros((16,), jnp.float32)

        def zrow(i, _):
            for kk in range(16):
                acc_v[i, pl.ds(kk * 16, 16)] = z
            return 0
        lax.fori_loop(0, WIDTH, zrow, 0, unroll=2)

        ys = (ya, yb, yc)
        for r in range(3):
            pltpu.sync_copy(bc.at[pl.ds((r * NT + wid) * 16, 16)], cnt_v)
            cnt = jnp.max(cnt_v[...])
            cnt16 = (cnt + 15) & ~15
            ngrp = (cnt16 + G - 1) // G
            pltpu.sync_copy(bs.at[pl.ds((r * NT + wid) * CAP, CAP)], s_v)
            pltpu.sync_copy(bd.at[pl.ds((r * NT + wid) * CAP, CAP)], d_v)
            pltpu.sync_copy(bww.at[pl.ds((r * NT + wid) * CAP, CAP)], w_v)

            def gcp(jg):
                idx = s_v.at[pl.ds(jg * G, G)]
                buf = rows_v.at[pl.ds(lax.rem(jg, 3) * G, G)]
                return pltpu.make_async_copy(ys[r].at[idx], buf, sem_g)

            @pl.when(ngrp > 0)
            def _():
                gcp(0).start()

            def grp(jg, _):
                @pl.when(jg + 1 < ngrp)
                def _():
                    gcp(jg + 1).start()
                gcp(jg).wait()
                par = lax.rem(jg, 3) * G
                for g in range(G // 16):
                    @pl.when(jg * G + g * 16 < cnt16)
                    def _():
                        w16 = w_v[pl.ds(jg * G + g * 16, 16)]
                        dl16 = d_v[pl.ds(jg * G + g * 16, 16)]
                        for e in range(16):
                            we = w16[e]
                            dle = dl16[e]
                            ro = par + g * 16 + e

                            @plsc.parallel_loop(0, 16, unroll=16)
                            def _(c):
                                plsc.addupdate(
                                    acc_v.at[dle, pl.ds(c * 16, 16)],
                                    rows_v[ro, pl.ds(c * 16, 16)] * we)
                return 0
            lax.fori_loop(0, ngrp, grp, 0)

        pltpu.sync_copy(acc_v, o.at[pl.ds(wid * WIDTH, WIDTH)])

    return k(y0, y1, y2, bsrc, bdl, bw, bcnt)


def _sc_score(h2_pre, t2, ps, pd, ns_, nd_):
    out_type = (
        jax.ShapeDtypeStruct((EP,), jnp.float32),
        jax.ShapeDtypeStruct((EP,), jnp.float32),
    )
    NG = EPS // GS

    @functools.partial(
        pl.kernel, mesh=_mesh(), out_type=out_type,
        compiler_params=_SC_PARAMS,
        scratch_types=[
            pltpu.VMEM((NPAD,), jnp.float32),
            pltpu.VMEM((2 * GS, D), jnp.float32),
            pltpu.VMEM((2 * GS, D), jnp.float32),
            pltpu.VMEM((EPS,), jnp.int32),
            pltpu.VMEM((EPS,), jnp.int32),
            pltpu.VMEM((EPS,), jnp.float32),
            pltpu.SemaphoreType.DMA,
            pltpu.SemaphoreType.DMA,
        ],
    )
    def k(h_hbm, t2_hbm, ps_h, pd_h, ns_h, nd_h, opos, oneg,
          t2_v, ur_v, vr_v, ui_v, vi_v, ob_v, sem_u, sem_v):
        wid = _wid()
        lanes = _LANES()
        pltpu.sync_copy(t2_hbm.at[pl.ds(0, NPAD)], t2_v)
        for (sref, dref, oref) in ((ps_h, pd_h, opos), (ns_h, nd_h, oneg)):
            pltpu.sync_copy(sref.at[pl.ds(wid * EPS, EPS)], ui_v)
            pltpu.sync_copy(dref.at[pl.ds(wid * EPS, EPS)], vi_v)

            def gcu(jg):
                return pltpu.make_async_copy(
                    h_hbm.at[ui_v.at[pl.ds(jg * GS, GS)]],
                    ur_v.at[pl.ds((jg & 1) * GS, GS)], sem_u)

            def gcv(jg):
                return pltpu.make_async_copy(
                    h_hbm.at[vi_v.at[pl.ds(jg * GS, GS)]],
                    vr_v.at[pl.ds((jg & 1) * GS, GS)], sem_v)

            gcu(0).start()
            gcv(0).start()

            def grp(jg, _):
                @pl.when(jg + 1 < NG)
                def _():
                    gcu(jg + 1).start()
                    gcv(jg + 1).start()
                gcu(jg).wait()
                gcv(jg).wait()
                par = (jg & 1) * GS
                for g in range(GS // 16):
                    u16 = ui_v[pl.ds(jg * GS + g * 16, 16)]
                    v16 = vi_v[pl.ds(jg * GS + g * 16, 16)]
                    acc0 = plsc.load_gather(t2_v, [u16]) + plsc.load_gather(t2_v, [v16])

                    @plsc.parallel_loop(0, 16, unroll=16, carry=acc0)
                    def acc(e, acc):
                        ro = par + g * 16 + e
                        pp = (ur_v[ro, pl.ds(0, 16)] * vr_v[ro, pl.ds(0, 16)])
                        for c in range(1, 16):
                            pp = pp + (ur_v[ro, pl.ds(c * 16, 16)]
                                       * vr_v[ro, pl.ds(c * 16, 16)])
                        return jnp.where(lanes == e, acc + jnp.sum(pp), acc)
                    ob_v[pl.ds(jg * GS + g * 16, 16)] = acc
                return 0
            lax.fori_loop(0, NG, grp, 0)
            pltpu.sync_copy(ob_v, oref.at[pl.ds(wid * EPS, EPS)])

    return k(h2_pre, t2, ps, pd, ns_, nd_)


def kernel(x, edge_index_r0, edge_index_r1, edge_index_r2, neg_edge_index, etype,
           W1_0, b1_0, W1_1, b1_1, W1_2, b1_2,
           W2_0, b2_0, W2_1, b2_1, W2_2, b2_2):
    e0 = edge_index_r0.astype(jnp.int32)
    e1 = edge_index_r1.astype(jnp.int32)
    e2 = edge_index_r2.astype(jnp.int32)
    ng = neg_edge_index.astype(jnp.int32)
    s0, d0 = e0[0], e0[1]
    s1, d1 = e1[0], e1[1]
    s2, d2 = e2[0], e2[1]

    xp = jnp.pad(x, ((0, NPAD - N), (0, 0)))
    zb = jnp.zeros((1, D), jnp.float32)
    b1sum = (b1_0 + b1_1 + b1_2)[None, :]
    b2sum = (b2_0 + b2_1 + b2_2)[None, :]

    y1_0, y1_1, y1_2 = _mm3(xp, W1_0, W1_1, W1_2, zb, relu=False)
    deg_part = _sc_degree(s0, d0, s1, d1, s2, d2).reshape(6, NT, NHIST)
    norms = _norms(deg_part).reshape(-1)
    bsrc, bdl, bw, bcnt = _sc_bin(s0, d0, s1, d1, s2, d2, norms)
    h_pre = _sc_spmm(y1_0, y1_1, y1_2, bsrc, bdl, bw, bcnt)
    y2_0, y2_1, y2_2 = _mm3(h_pre, W2_0, W2_1, W2_2, b1sum, relu=True)
    h2_pre = _sc_spmm(y2_0, y2_1, y2_2, bsrc, bdl, bw, bcnt)
    t2 = _t2(h2_pre, b2sum).reshape(-1)
    pe = jnp.stack([e0, e1, e2])[jnp.asarray(etype)]
    pad = lambda a: jnp.pad(a, (0, EP - E))
    pos_f, neg_f = _sc_score(h2_pre, t2, pad(pe[0]), pad(pe[1]),
                             pad(ng[0]), pad(ng[1]))
    return pos_f[:E, None], neg_f[:E, None]


# speedup vs baseline: 1.0095x; 1.0095x over previous
import functools

import jax
import jax.numpy as jnp
from jax import lax
from jax.experimental import pallas as pl
from jax.experimental.pallas import tpu as pltpu
from jax.experimental.pallas import tpu_sc as plsc

N = 10000
D = 256
E = 160000
NT = 32
WIDTH = 320
NPAD = NT * WIDTH
CAP = 6144
NHIST = 10240
ECH = 2000
EPT = E // NT
EP = 163840
EPS = EP // NT
G = 32
GS = 32
NBUF = 4

_SC_PARAMS = pltpu.CompilerParams(needs_layout_passes=False)


def _mesh():
    return plsc.VectorSubcoreMesh(core_axis_name="c", subcore_axis_name="s")


def _wid():
    return lax.axis_index("s") * 2 + lax.axis_index("c")


_LANES = lambda: lax.iota(jnp.int32, 16)


def _mm3_body(a_ref, w0, w1, w2, b, o0, o1, o2, *, relu):
    t = a_ref[...] + b[...]
    if relu:
        t = jnp.maximum(t, 0.0)
    o0[...] = jnp.dot(t, w0[...], preferred_element_type=jnp.float32)
    o1[...] = jnp.dot(t, w1[...], preferred_element_type=jnp.float32)
    o2[...] = jnp.dot(t, w2[...], preferred_element_type=jnp.float32)


def _mm3(a, w0, w1, w2, b, relu):
    rows = NPAD // 8
    blk = pl.BlockSpec((rows, D), lambda i: (i, 0))
    wspec = pl.BlockSpec((D, D), lambda i: (0, 0))
    bspec = pl.BlockSpec((1, D), lambda i: (0, 0))
    sds = jax.ShapeDtypeStruct((NPAD, D), jnp.float32)
    return pl.pallas_call(
        functools.partial(_mm3_body, relu=relu),
        grid=(8,),
        in_specs=[blk, wspec, wspec, wspec, bspec],
        out_specs=(blk, blk, blk),
        out_shape=(sds, sds, sds),
    )(a, w0, w1, w2, b)


def _norm_body(dpart_ref, o_ref):
    deg = jnp.sum(dpart_ref[0], axis=0)
    nrm = jnp.where(deg > 0, lax.rsqrt(deg), 0.0)
    o_ref[...] = jnp.broadcast_to(nrm[None, :], o_ref.shape)


def _norms(deg_part):
    return pl.pallas_call(
        _norm_body,
        grid=(6,),
        in_specs=[pl.BlockSpec((1, NT, NHIST), lambda i: (i, 0, 0))],
        out_specs=pl.BlockSpec((8, NHIST), lambda i: (i, 0)),
        out_shape=jax.ShapeDtypeStruct((48, NHIST), jnp.float32),
    )(deg_part)


def _t2_body(h_ref, b_ref, o_ref):
    b = b_ref[...]
    t2 = jnp.sum(h_ref[...] * b, axis=1) + 0.5 * jnp.sum(b * b)
    o_ref[...] = jnp.broadcast_to(t2[None, :], o_ref.shape)


def _t2(h2_pre, b2sum):
    return pl.pallas_call(
        _t2_body,
        out_shape=jax.ShapeDtypeStruct((8, NPAD), jnp.float32),
    )(h2_pre, b2sum)


def _sc_degree(s0, d0, s1, d1, s2, d2):
    out_type = jax.ShapeDtypeStruct((6 * NT * NHIST,), jnp.float32)

    @functools.partial(
        pl.kernel, mesh=_mesh(), out_type=out_type,
        compiler_params=_SC_PARAMS,
        scratch_types=[
            pltpu.VMEM((NHIST,), jnp.float32),
            pltpu.VMEM((5008,), jnp.int32),
        ],
    )
    def k(e0, e1, e2, e3, e4, e5, o, hist_v, idx_v):
        wid = _wid()
        lanes = _LANES()
        ones = jnp.ones((16,), jnp.float32)
        z = jnp.zeros((16,), jnp.float32)
        for ai, eref in enumerate((e0, e1, e2, e3, e4, e5)):
            def zbody(i, _):
                hist_v[pl.ds(i * 16, 16)] = z
                return 0
            lax.fori_loop(0, NHIST // 16, zbody, 0, unroll=8)
            pltpu.sync_copy(eref.at[pl.ds(wid * EPT, EPT)], idx_v.at[pl.ds(0, EPT)])

            def hbody(i, _):
                v = idx_v[pl.ds(i * 16, 16)]
                plsc.addupdate_scatter(hist_v, [v], ones)
                return 0
            lax.fori_loop(0, EPT // 16, hbody, 0, unroll=8)
            vt = idx_v[pl.ds((EPT // 16) * 16, 16)]
            plsc.addupdate_scatter(hist_v, [vt], ones, mask=lanes < (EPT % 16))
            pltpu.sync_copy(hist_v, o.at[pl.ds((ai * NT + wid) * NHIST, NHIST)])

    return k(s0, d0, s1, d1, s2, d2)


def _sc_bin(s0, d0, s1, d1, s2, d2, norms):
    out_type = (
        jax.ShapeDtypeStruct((3 * NT * CAP,), jnp.int32),
        jax.ShapeDtypeStruct((3 * NT * CAP,), jnp.int32),
        jax.ShapeDtypeStruct((3 * NT * CAP,), jnp.float32),
        jax.ShapeDtypeStruct((3 * NT * 16,), jnp.int32),
    )

    @functools.partial(
        pl.kernel, mesh=_mesh(), out_type=out_type,
        compiler_params=_SC_PARAMS,
        scratch_types=[
            pltpu.VMEM((NHIST,), jnp.float32),
            pltpu.VMEM((NHIST,), jnp.float32),
            pltpu.VMEM((ECH,), jnp.int32),
            pltpu.VMEM((ECH,), jnp.int32),
            pltpu.VMEM((CAP + 16,), jnp.int32),
            pltpu.VMEM((CAP + 16,), jnp.int32),
            pltpu.VMEM((CAP + 16,), jnp.float32),
            pltpu.VMEM((16,), jnp.int32),
        ],
    )
    def k(es0, ed0, es1, ed1, es2, ed2, nrm,
          bsrc, bdl, bw, bcnt,
          ns_v, nd_v, sc_v, dc_v, os_v, od_v, ow_v, cnt_v):
        wid = _wid()
        zi = jnp.zeros((16,), jnp.int32)
        zf = jnp.zeros((16,), jnp.float32)
        srcs = (es0, es1, es2)
        dsts = (ed0, ed1, ed2)
        for r in range(3):
            pltpu.sync_copy(nrm.at[pl.ds((2 * r) * 8 * NHIST, NHIST)], ns_v)
            pltpu.sync_copy(nrm.at[pl.ds((2 * r + 1) * 8 * NHIST, NHIST)], nd_v)

            def zbody(i, _):
                os_v[pl.ds(i * 16, 16)] = zi
                od_v[pl.ds(i * 16, 16)] = zi
                ow_v[pl.ds(i * 16, 16)] = zf
                return 0
            lax.fori_loop(0, (CAP + 16) // 16, zbody, 0, unroll=8)

            def chunk(kk, cnt):
                pltpu.sync_copy(srcs[r].at[pl.ds(kk * ECH, ECH)], sc_v)
                pltpu.sync_copy(dsts[r].at[pl.ds(kk * ECH, ECH)], dc_v)

                def vbody(i, cnt):
                    d_v = dc_v[pl.ds(i * 16, 16)]
                    s_v = sc_v[pl.ds(i * 16, 16)]
                    q = (d_v * 3277) >> 20
                    q = jnp.where(q * WIDTH > d_v, q - 1, q)
                    m = (q == wid) & (cnt < CAP - 32)
                    dl = d_v - wid * WIDTH
                    w = plsc.load_gather(ns_v, [s_v]) * plsc.load_gather(nd_v, [d_v])
                    plsc.store_compressed(os_v.at[pl.ds(cnt, 16)], s_v, mask=m)
                    plsc.store_compressed(od_v.at[pl.ds(cnt, 16)], dl, mask=m)
                    plsc.store_compressed(ow_v.at[pl.ds(cnt, 16)], w, mask=m)
                    return cnt + plsc.all_reduce_population_count(m)[0]
                return lax.fori_loop(0, ECH // 16, vbody, cnt)

            cnt = lax.fori_loop(0, E // ECH, chunk, jnp.int32(0))
            os_v[pl.ds(cnt, 16)] = zi
            od_v[pl.ds(cnt, 16)] = zi
            ow_v[pl.ds(cnt, 16)] = zf
            pltpu.sync_copy(os_v.at[pl.ds(0, CAP)],
                            bsrc.at[pl.ds((r * NT + wid) * CAP, CAP)])
            pltpu.sync_copy(od_v.at[pl.ds(0, CAP)],
                            bdl.at[pl.ds((r * NT + wid) * CAP, CAP)])
            pltpu.sync_copy(ow_v.at[pl.ds(0, CAP)],
                            bw.at[pl.ds((r * NT + wid) * CAP, CAP)])
            cnt_v[...] = jnp.broadcast_to(cnt, (16,))
            pltpu.sync_copy(cnt_v, bcnt.at[pl.ds((r * NT + wid) * 16, 16)])

    return k(s0, d0, s1, d1, s2, d2, norms)


def _sc_spmm(y0, y1, y2, bsrc, bdl, bw, bcnt):
    out_type = jax.ShapeDtypeStruct((NPAD, D), jnp.float32)

    @functools.partial(
        pl.kernel, mesh=_mesh(), out_type=out_type,
        compiler_params=_SC_PARAMS,
        scratch_types=[
            pltpu.VMEM((WIDTH, D), jnp.float32),
            pltpu.VMEM((3 * G, D), jnp.float32),
            pltpu.VMEM((CAP,), jnp.int32),
            pltpu.VMEM((CAP,), jnp.int32),
            pltpu.VMEM((CAP,), jnp.float32),
            pltpu.VMEM((16,), jnp.int32),
            pltpu.SemaphoreType.DMA,
        ],
    )
    def k(ya, yb, yc, bs, bd, bww, bc, o,
          acc_v, rows_v, s_v, d_v, w_v, cnt_v, sem_g):
        wid = _wid()
        z = jnp.zeros((16,), jnp.float32)

        def zrow(i, _):
            for kk in range(16):
                acc_v[i, pl.ds(kk * 16, 16)] = z
            return 0
        lax.fori_loop(0, WIDTH, zrow, 0, unroll=2)

        ys = (ya, yb, yc)
        for r in range(3):
            pltpu.sync_copy(bc.at[pl.ds((r * NT + wid) * 16, 16)], cnt_v)
            cnt = jnp.max(cnt_v[...])
            cnt16 = (cnt + 15) & ~15
            ngrp = (cnt16 + G - 1) // G
            pltpu.sync_copy(bs.at[pl.ds((r * NT + wid) * CAP, CAP)], s_v)
            pltpu.sync_copy(bd.at[pl.ds((r * NT + wid) * CAP, CAP)], d_v)
            pltpu.sync_copy(bww.at[pl.ds((r * NT + wid) * CAP, CAP)], w_v)

            def gcp(jg):
                idx = s_v.at[pl.ds(jg * G, G)]
                buf = rows_v.at[pl.ds(lax.rem(jg, 3) * G, G)]
                return pltpu.make_async_copy(ys[r].at[idx], buf, sem_g)

            @pl.when(ngrp > 0)
            def _():
                gcp(0).start()

            def grp(jg, _):
                @pl.when(jg + 1 < ngrp)
                def _():
                    gcp(jg + 1).start()
                gcp(jg).wait()
                par = lax.rem(jg, 3) * G
                for g in range(G // 16):
                    @pl.when(jg * G + g * 16 < cnt16)
                    def _():
                        w16 = w_v[pl.ds(jg * G + g * 16, 16)]
                        dl16 = d_v[pl.ds(jg * G + g * 16, 16)]
                        for e in range(16):
                            we = w16[e]
                            dle = dl16[e]
                            ro = par + g * 16 + e

                            @plsc.parallel_loop(0, 16, unroll=16)
                            def _(c):
                                plsc.addupdate(
                                    acc_v.at[dle, pl.ds(c * 16, 16)],
                                    rows_v[ro, pl.ds(c * 16, 16)] * we)
                return 0
            lax.fori_loop(0, ngrp, grp, 0)

        pltpu.sync_copy(acc_v, o.at[pl.ds(wid * WIDTH, WIDTH)])

    return k(y0, y1, y2, bsrc, bdl, bw, bcnt)


def _sc_score(h2_pre, t2, ps, pd, ns_, nd_):
    out_type = (
        jax.ShapeDtypeStruct((EP,), jnp.float32),
        jax.ShapeDtypeStruct((EP,), jnp.float32),
    )
    NG = EPS // GS

    @functools.partial(
        pl.kernel, mesh=_mesh(), out_type=out_type,
        compiler_params=_SC_PARAMS,
        scratch_types=[
            pltpu.VMEM((NPAD,), jnp.float32),
            pltpu.VMEM((NBUF * GS, D), jnp.float32),
            pltpu.VMEM((NBUF * GS, D), jnp.float32),
            pltpu.VMEM((EPS,), jnp.int32),
            pltpu.VMEM((EPS,), jnp.int32),
            pltpu.VMEM((EPS,), jnp.float32),
            pltpu.SemaphoreType.DMA,
            pltpu.SemaphoreType.DMA,
        ],
    )
    def k(h_hbm, t2_hbm, ps_h, pd_h, ns_h, nd_h, opos, oneg,
          t2_v, ur_v, vr_v, ui_v, vi_v, ob_v, sem_u, sem_v):
        wid = _wid()
        lanes = _LANES()
        pltpu.sync_copy(t2_hbm.at[pl.ds(0, NPAD)], t2_v)
        for (sref, dref, oref) in ((ps_h, pd_h, opos), (ns_h, nd_h, oneg)):
            pltpu.sync_copy(sref.at[pl.ds(wid * EPS, EPS)], ui_v)
            pltpu.sync_copy(dref.at[pl.ds(wid * EPS, EPS)], vi_v)

            def gcu(jg):
                return pltpu.make_async_copy(
                    h_hbm.at[ui_v.at[pl.ds(jg * GS, GS)]],
                    ur_v.at[pl.ds(lax.rem(jg, NBUF) * GS, GS)], sem_u)

            def gcv(jg):
                return pltpu.make_async_copy(
                    h_hbm.at[vi_v.at[pl.ds(jg * GS, GS)]],
                    vr_v.at[pl.ds(lax.rem(jg, NBUF) * GS, GS)], sem_v)

            for j0 in range(NBUF - 1):
                gcu(j0).start()
                gcv(j0).start()

            def grp(jg, _):
                @pl.when(jg + NBUF - 1 < NG)
                def _():
                    gcu(jg + NBUF - 1).start()
                    gcv(jg + NBUF - 1).start()
                gcu(jg).wait()
                gcv(jg).wait()
                par = lax.rem(jg, NBUF) * GS
                for g in range(GS // 16):
                    u16 = ui_v[pl.ds(jg * GS + g * 16, 16)]
                    v16 = vi_v[pl.ds(jg * GS + g * 16, 16)]
                    acc0 = plsc.load_gather(t2_v, [u16]) + plsc.load_gather(t2_v, [v16])

                    @plsc.parallel_loop(0, 16, unroll=16, carry=acc0)
                    def acc(e, acc):
                        ro = par + g * 16 + e
                        pp = (ur_v[ro, pl.ds(0, 16)] * vr_v[ro, pl.ds(0, 16)])
                        for c in range(1, 16):
                            pp = pp + (ur_v[ro, pl.ds(c * 16, 16)]
                                       * vr_v[ro, pl.ds(c * 16, 16)])
                        return jnp.where(lanes == e, acc + jnp.sum(pp), acc)
                    ob_v[pl.ds(jg * GS + g * 16, 16)] = acc
                return 0
            lax.fori_loop(0, NG, grp, 0)
            pltpu.sync_copy(ob_v, oref.at[pl.ds(wid * EPS, EPS)])

    return k(h2_pre, t2, ps, pd, ns_, nd_)


def kernel(x, edge_index_r0, edge_index_r1, edge_index_r2, neg_edge_index, etype,
           W1_0, b1_0, W1_1, b1_1, W1_2, b1_2,
           W2_0, b2_0, W2_1, b2_1, W2_2, b2_2):
    e0 = edge_index_r0.astype(jnp.int32)
    e1 = edge_index_r1.astype(jnp.int32)
    e2 = edge_index_r2.astype(jnp.int32)
    ng = neg_edge_index.astype(jnp.int32)
    s0, d0 = e0[0], e0[1]
    s1, d1 = e1[0], e1[1]
    s2, d2 = e2[0], e2[1]

    xp = jnp.pad(x, ((0, NPAD - N), (0, 0)))
    zb = jnp.zeros((1, D), jnp.float32)
    b1sum = (b1_0 + b1_1 + b1_2)[None, :]
    b2sum = (b2_0 + b2_1 + b2_2)[None, :]

    y1_0, y1_1, y1_2 = _mm3(xp, W1_0, W1_1, W1_2, zb, relu=False)
    deg_part = _sc_degree(s0, d0, s1, d1, s2, d2).reshape(6, NT, NHIST)
    norms = _norms(deg_part).reshape(-1)
    bsrc, bdl, bw, bcnt = _sc_bin(s0, d0, s1, d1, s2, d2, norms)
    h_pre = _sc_spmm(y1_0, y1_1, y1_2, bsrc, bdl, bw, bcnt)
    y2_0, y2_1, y2_2 = _mm3(h_pre, W2_0, W2_1, W2_2, b1sum, relu=True)
    h2_pre = _sc_spmm(y2_0, y2_1, y2_2, bsrc, bdl, bw, bcnt)
    t2 = _t2(h2_pre, b2sum).reshape(-1)
    pe = jnp.stack([e0, e1, e2])[jnp.asarray(etype)]
    pad = lambda a: jnp.pad(a, (0, EP - E))
    pos_f, neg_f = _sc_score(h2_pre, t2, pad(pe[0]), pad(pe[1]),
                             pad(ng[0]), pad(ng[1]))
    return pos_f[:E, None], neg_f[:E, None]
